# Initial kernel scaffold; baseline (speedup 1.0000x reference)
import jax
import jax.numpy as jnp
from jax.experimental import pallas as pl


def kernel(inputs, pos_table):
    raise NotImplementedError("write your pallas kernel here")


# speedup vs baseline: 1.4884x; 1.4884x over previous
import jax
import jax.numpy as jnp
from jax.experimental import pallas as pl

_SEQ_BLOCK = 512


def _add_kernel(x_ref, p_ref, o_ref):
    o_ref[...] = x_ref[...] + p_ref[...]


def kernel(inputs, pos_table):
    B, S, D = inputs.shape
    n_seq = S // _SEQ_BLOCK
    return pl.pallas_call(
        _add_kernel,
        grid=(n_seq, B),
        in_specs=[
            pl.BlockSpec((1, _SEQ_BLOCK, D), lambda s, b: (b, s, 0)),
            pl.BlockSpec((_SEQ_BLOCK, D), lambda s, b: (s, 0)),
        ],
        out_specs=pl.BlockSpec((1, _SEQ_BLOCK, D), lambda s, b: (b, s, 0)),
        out_shape=jax.ShapeDtypeStruct((B, S, D), inputs.dtype),
    )(inputs, pos_table)
